# Initial kernel scaffold; baseline (speedup 1.0000x reference)
import jax
import jax.numpy as jnp
from jax.experimental import pallas as pl


def kernel(x, W_enc, b_enc, W_dec, bias):
    raise NotImplementedError("write your pallas kernel here")


# speedup vs baseline: 11.8448x; 11.8448x over previous
import functools

import jax
import jax.numpy as jnp
from jax.experimental import pallas as pl

INPUT_DIM = 2048
HIDDEN_DIM = 16384
BATCH = 8192
K = 64


def _enc_body(x_ref, w_ref, b_ref, out_ref):
    acc = jax.lax.dot_general(
        x_ref[...], w_ref[...],
        dimension_numbers=(((1,), (1,)), ((), ())),
        preferred_element_type=jnp.float32)
    out_ref[...] = jnp.maximum(acc + b_ref[...], 0.0)


def _encode(xcb, Wb, b2):
    R = 512
    C = 2048
    grid = (HIDDEN_DIM // C, BATCH // R)
    return pl.pallas_call(
        _enc_body,
        grid=grid,
        in_specs=[
            pl.BlockSpec((R, INPUT_DIM), lambda c, r: (r, 0)),
            pl.BlockSpec((C, INPUT_DIM), lambda c, r: (c, 0)),
            pl.BlockSpec((1, C), lambda c, r: (0, c)),
        ],
        out_specs=pl.BlockSpec((R, C), lambda c, r: (r, c)),
        out_shape=jax.ShapeDtypeStruct((BATCH, HIDDEN_DIM), jnp.float32),
    )(xcb, Wb, b2)


def _topk_body(lat_ref, out_ref):
    lat = lat_ref[...]
    li = jax.lax.bitcast_convert_type(lat, jnp.int32)
    rows = lat.shape[0]

    def step(i, T):
        b = 30 - i
        cand = T | (1 << b)
        cnt = jnp.sum((li >= cand).astype(jnp.int32), axis=1, keepdims=True)
        return jnp.where(cnt >= K, cand, T)

    T0 = jnp.zeros((rows, 1), jnp.int32)
    T = jax.lax.fori_loop(0, 31, step, T0)
    out_ref[...] = jnp.where(li >= T, lat, 0.0)


def _topk_mask(latent):
    R = 128
    return pl.pallas_call(
        _topk_body,
        grid=(BATCH // R,),
        in_specs=[pl.BlockSpec((R, HIDDEN_DIM), lambda r: (r, 0))],
        out_specs=pl.BlockSpec((R, HIDDEN_DIM), lambda r: (r, 0)),
        out_shape=jax.ShapeDtypeStruct((BATCH, HIDDEN_DIM), jnp.float32),
    )(latent)


def _dec_body(lat_ref, w_ref, b_ref, out_ref):
    k = pl.program_id(1)

    @pl.when(k == 0)
    def _():
        out_ref[...] = jnp.broadcast_to(b_ref[...], out_ref.shape)

    lat_bf = lat_ref[...].astype(jnp.bfloat16)
    out_ref[...] += jax.lax.dot_general(
        lat_bf, w_ref[...],
        dimension_numbers=(((1,), (1,)), ((), ())),
        preferred_element_type=jnp.float32)


def _decode(latent_sparse, Wdb, bias2):
    G = 1024
    Kc = 2048
    grid = (BATCH // G, HIDDEN_DIM // Kc)
    return pl.pallas_call(
        _dec_body,
        grid=grid,
        in_specs=[
            pl.BlockSpec((G, Kc), lambda g, k: (g, k)),
            pl.BlockSpec((INPUT_DIM, Kc), lambda g, k: (0, k)),
            pl.BlockSpec((1, INPUT_DIM), lambda g, k: (0, 0)),
        ],
        out_specs=pl.BlockSpec((G, INPUT_DIM), lambda g, k: (g, 0)),
        out_shape=jax.ShapeDtypeStruct((BATCH, INPUT_DIM), jnp.float32),
    )(latent_sparse, Wdb, bias2)


def kernel(x, W_enc, b_enc, W_dec, bias):
    xcb = (x - bias).astype(jnp.bfloat16)
    Wb = W_enc.astype(jnp.bfloat16)
    latent = _encode(xcb, Wb, b_enc.reshape(1, -1))
    latent_sparse = _topk_mask(latent)
    recon = _decode(latent_sparse, W_dec.astype(jnp.bfloat16),
                    bias.reshape(1, -1))
    return (latent_sparse, recon)
